# baseline (speedup 1.0000x reference)
import functools

import jax
import jax.numpy as jnp
from jax import lax
from jax.experimental import pallas as pl
from jax.experimental.pallas import tpu as pltpu
from jax.experimental.pallas import tpu_sc as plsc

IDIM = 128
HDIM = 64
KNN = 16
NB = 2
NPTS = 8192
C1 = 64
C2 = 64
D2 = 2 * HDIM

_NW = 32
_CH = 128


def _sc_gather(table, idx):
    rows = idx.shape[0]
    width = table.shape[1]
    rows_pw = rows // _NW
    nchunk = rows_pw // _CH
    mesh = plsc.VectorSubcoreMesh(core_axis_name="c", subcore_axis_name="s")

    @functools.partial(
        pl.kernel,
        mesh=mesh,
        compiler_params=pltpu.CompilerParams(use_tc_tiling_on_sc=False),
        out_type=jax.ShapeDtypeStruct((rows, width), table.dtype),
        scratch_types=[
            pltpu.VMEM((_CH,), jnp.int32),
            pltpu.VMEM((_CH, width), table.dtype),
            pltpu.SemaphoreType.DMA,
        ],
    )
    def gk(table_hbm, idx_hbm, out_hbm, idx_v, rows_v, sem):
        wid = lax.axis_index("s") * 2 + lax.axis_index("c")
        base = wid * rows_pw

        def body(c, carry):
            off = base + c * _CH
            pltpu.sync_copy(idx_hbm.at[pl.ds(off, _CH)], idx_v)
            pltpu.async_copy(table_hbm.at[idx_v], rows_v, sem).wait()
            pltpu.sync_copy(rows_v, out_hbm.at[pl.ds(off, _CH)])
            return carry

        lax.fori_loop(0, nchunk, body, 0)

    return gk(table, idx)


_T1 = 2048
_T = 256


def _p1_body(x_ref, logs_ref, bias_ref, w_ref, b1_ref, idx_ref,
             z_ref, a_ref, bm_ref, idxo_ref):
    b = pl.program_id(0)
    z = x_ref[0] * jnp.exp(logs_ref[0]) + bias_ref[0]
    z_ref[0] = z
    cat = jnp.dot(z[:, :C1], w_ref[...], preferred_element_type=jnp.float32)
    a_ref[0] = cat[:, :D2] + b1_ref[...]
    v = cat[:, D2:].astype(jnp.bfloat16).astype(jnp.float32)
    u = lax.bitcast_convert_type(v, jnp.uint32)
    packed = (u[:, C1:] & jnp.uint32(0xFFFF0000)) | (u[:, :C1] >> 16)
    bm_ref[0] = lax.bitcast_convert_type(packed, jnp.int32)
    idxo_ref[0] = idx_ref[0] + b * NPTS


def _coupling_core(g_ref, a_ref, z_ref, w2_ref, b2_ref, w3_ref, b3_ref):
    u = lax.bitcast_convert_type(g_ref[0], jnp.uint32)
    g_lo = lax.bitcast_convert_type(u << 16, jnp.float32)
    g_hi = lax.bitcast_convert_type(u & jnp.uint32(0xFFFF0000), jnp.float32)
    g = jnp.concatenate([g_lo, g_hi], axis=1)
    a = a_ref[0]
    h1 = jax.nn.relu(g.reshape(_T, KNN, D2) + a[:, None, :]).reshape(_T * KNN, D2)
    h2 = jax.nn.relu(
        jnp.dot(h1, w2_ref[...], preferred_element_type=jnp.float32) + b2_ref[...])
    m = jnp.max(h2.reshape(_T, KNN, D2), axis=1)
    o = jnp.dot(m, w3_ref[...], preferred_element_type=jnp.float32) + b3_ref[...]
    sl = o[:, :C2] + 2.0
    s = jax.nn.sigmoid(sl)
    z = z_ref[0]
    xb2 = (z[:, C1:] + o[:, C2:]) * s
    z1 = jnp.concatenate([z[:, :C1], xb2], axis=1)
    part = -jnp.sum(jnp.maximum(-sl, 0.0) + jnp.log1p(jnp.exp(-jnp.abs(sl))))
    return z1, part


def _p3_body(g_ref, a_ref, z_ref, logs_ref, w2_ref, b2_ref, w3_ref, b3_ref,
             p_ref, w1n_ref, b1n_ref, zp_ref, a2_ref, bm2_ref, ld_ref):
    t_idx = pl.program_id(1)
    z1, part = _coupling_core(g_ref, a_ref, z_ref, w2_ref, b2_ref, w3_ref, b3_ref)
    zp = jnp.dot(z1, p_ref[...], preferred_element_type=jnp.float32)
    zp_ref[0] = zp
    cat = jnp.dot(zp[:, :C1], w1n_ref[...], preferred_element_type=jnp.float32)
    a2_ref[0] = cat[:, :D2] + b1n_ref[...]
    v = cat[:, D2:].astype(jnp.bfloat16).astype(jnp.float32)
    u2 = lax.bitcast_convert_type(v, jnp.uint32)
    packed = (u2[:, C1:] & jnp.uint32(0xFFFF0000)) | (u2[:, :C1] >> 16)
    bm2_ref[0] = lax.bitcast_convert_type(packed, jnp.int32)

    @pl.when(t_idx == 0)
    def _():
        ld_ref[0, 0, 0] = jnp.sum(logs_ref[0]) * NPTS + part

    @pl.when(t_idx != 0)
    def _():
        ld_ref[0, 0, 0] += part


def _p5_body(g_ref, a_ref, z_ref, ld3_ref, w2_ref, b2_ref, w3_ref, b3_ref,
             zf_ref, ld_ref):
    t_idx = pl.program_id(1)
    z1, part = _coupling_core(g_ref, a_ref, z_ref, w2_ref, b2_ref, w3_ref, b3_ref)
    zf_ref[0] = z1

    @pl.when(t_idx == 0)
    def _():
        ld_ref[0, 0, 0] = ld3_ref[0, 0, 0] + part

    @pl.when(t_idx != 0)
    def _():
        ld_ref[0, 0, 0] += part


def _fold_l1(p):
    w = p["l1"]["w"]
    wc, wd, wn = w[:C1], w[C1:2 * C1], w[2 * C1:]
    return wc - wd, wd + wn, p["l1"]["b"]


def _pack_coupling(pF, pG):
    waF, wbF, b1F = _fold_l1(pF)
    waG, wbG, b1G = _fold_l1(pG)
    w1 = jnp.concatenate(
        [waF, waG, wbF, wbG], axis=1)
    b1 = jnp.concatenate([b1F, b1G])[None, :]
    w2 = jnp.zeros((D2, D2), jnp.float32)
    w2 = w2.at[:HDIM, :HDIM].set(pF["l2"]["w"]).at[HDIM:, HDIM:].set(pG["l2"]["w"])
    b2 = jnp.concatenate([pF["l2"]["b"], pG["l2"]["b"]])[None, :]
    w3 = jnp.zeros((D2, 2 * C2), jnp.float32)
    w3 = w3.at[:HDIM, :C2].set(pF["l3"]["w"]).at[HDIM:, C2:].set(pG["l3"]["w"])
    b3 = jnp.concatenate([pF["l3"]["b"], pG["l3"]["b"]])[None, :]
    return w1, b1, w2, b2, w3, b3


def kernel(x, knn_idx, params, perm):
    x = x.astype(jnp.float32)
    idx3 = knn_idx.astype(jnp.int32).reshape(NB, (NPTS * KNN) // 128, 128)
    logs = params["actnorm"]["logs"]
    bias = params["actnorm"]["bias"]
    w1c1, b1c1, w2c1, b2c1, w3c1, b3c1 = _pack_coupling(params["c1F"], params["c1G"])
    w1c2, b1c2, w2c2, b2c2, w3c2, b3c2 = _pack_coupling(params["c2F"], params["c2G"])
    pmat = (jnp.arange(IDIM)[:, None] == perm[None, :].astype(jnp.int32)
            ).astype(jnp.float32)

    nt1 = NPTS // _T1
    it1 = (NPTS * KNN) // 128 // nt1
    f32 = jnp.float32
    cp = pltpu.CompilerParams(dimension_semantics=("parallel", "arbitrary"))

    z0, a1, bm1, idxo = pl.pallas_call(
        _p1_body,
        grid=(NB, nt1),
        in_specs=[
            pl.BlockSpec((1, _T1, IDIM), lambda b, t: (b, t, 0)),
            pl.BlockSpec((1, 1, IDIM), lambda b, t: (0, 0, 0)),
            pl.BlockSpec((1, 1, IDIM), lambda b, t: (0, 0, 0)),
            pl.BlockSpec((C1, 2 * D2), lambda b, t: (0, 0)),
            pl.BlockSpec((1, D2), lambda b, t: (0, 0)),
            pl.BlockSpec((1, it1, 128), lambda b, t: (b, t, 0)),
        ],
        out_specs=[
            pl.BlockSpec((1, _T1, IDIM), lambda b, t: (b, t, 0)),
            pl.BlockSpec((1, _T1, D2), lambda b, t: (b, t, 0)),
            pl.BlockSpec((1, _T1, C1), lambda b, t: (b, t, 0)),
            pl.BlockSpec((1, it1, 128), lambda b, t: (b, t, 0)),
        ],
        out_shape=[
            jax.ShapeDtypeStruct((NB, NPTS, IDIM), f32),
            jax.ShapeDtypeStruct((NB, NPTS, D2), f32),
            jax.ShapeDtypeStruct((NB, NPTS, C1), jnp.int32),
            jax.ShapeDtypeStruct((NB, (NPTS * KNN) // 128, 128), jnp.int32),
        ],
        compiler_params=cp,
    )(x, logs, bias, w1c1, b1c1, idx3)

    idx_flat = idxo.reshape(NB * NPTS * KNN)
    g1 = _sc_gather(bm1.reshape(NB * NPTS, C1), idx_flat)
    g1 = g1.reshape(NB, NPTS * KNN, C1)

    nt = NPTS // _T
    zp, a2, bm2, ld3 = pl.pallas_call(
        _p3_body,
        grid=(NB, nt),
        in_specs=[
            pl.BlockSpec((1, _T * KNN, C1), lambda b, t: (b, t, 0)),
            pl.BlockSpec((1, _T, D2), lambda b, t: (b, t, 0)),
            pl.BlockSpec((1, _T, IDIM), lambda b, t: (b, t, 0)),
            pl.BlockSpec((1, 1, IDIM), lambda b, t: (0, 0, 0)),
            pl.BlockSpec((D2, D2), lambda b, t: (0, 0)),
            pl.BlockSpec((1, D2), lambda b, t: (0, 0)),
            pl.BlockSpec((D2, 2 * C2), lambda b, t: (0, 0)),
            pl.BlockSpec((1, 2 * C2), lambda b, t: (0, 0)),
            pl.BlockSpec((IDIM, IDIM), lambda b, t: (0, 0)),
            pl.BlockSpec((C1, 2 * D2), lambda b, t: (0, 0)),
            pl.BlockSpec((1, D2), lambda b, t: (0, 0)),
        ],
        out_specs=[
            pl.BlockSpec((1, _T, IDIM), lambda b, t: (b, t, 0)),
            pl.BlockSpec((1, _T, D2), lambda b, t: (b, t, 0)),
            pl.BlockSpec((1, _T, C1), lambda b, t: (b, t, 0)),
            pl.BlockSpec((1, 1, 1), lambda b, t: (b, 0, 0), memory_space=pltpu.SMEM),
        ],
        out_shape=[
            jax.ShapeDtypeStruct((NB, NPTS, IDIM), f32),
            jax.ShapeDtypeStruct((NB, NPTS, D2), f32),
            jax.ShapeDtypeStruct((NB, NPTS, C1), jnp.int32),
            jax.ShapeDtypeStruct((NB, 1, 1), f32),
        ],
        compiler_params=cp,
    )(g1, a1, z0, logs, w2c1, b2c1, w3c1, b3c1, pmat, w1c2, b1c2)

    g2 = _sc_gather(bm2.reshape(NB * NPTS, C1), idx_flat)
    g2 = g2.reshape(NB, NPTS * KNN, C1)

    zf, ld = pl.pallas_call(
        _p5_body,
        grid=(NB, nt),
        in_specs=[
            pl.BlockSpec((1, _T * KNN, C1), lambda b, t: (b, t, 0)),
            pl.BlockSpec((1, _T, D2), lambda b, t: (b, t, 0)),
            pl.BlockSpec((1, _T, IDIM), lambda b, t: (b, t, 0)),
            pl.BlockSpec((1, 1, 1), lambda b, t: (b, 0, 0), memory_space=pltpu.SMEM),
            pl.BlockSpec((D2, D2), lambda b, t: (0, 0)),
            pl.BlockSpec((1, D2), lambda b, t: (0, 0)),
            pl.BlockSpec((D2, 2 * C2), lambda b, t: (0, 0)),
            pl.BlockSpec((1, 2 * C2), lambda b, t: (0, 0)),
        ],
        out_specs=[
            pl.BlockSpec((1, _T, IDIM), lambda b, t: (b, t, 0)),
            pl.BlockSpec((1, 1, 1), lambda b, t: (b, 0, 0), memory_space=pltpu.SMEM),
        ],
        out_shape=[
            jax.ShapeDtypeStruct((NB, NPTS, IDIM), f32),
            jax.ShapeDtypeStruct((NB, 1, 1), f32),
        ],
        compiler_params=cp,
    )(g2, a2, zp, ld3, w2c2, b2c2, w3c2, b3c2)

    return zf, ld[:, 0, 0]


# speedup vs baseline: 1.8147x; 1.8147x over previous
import functools

import jax
import jax.numpy as jnp
from jax import lax
from jax.experimental import pallas as pl
from jax.experimental.pallas import tpu as pltpu
from jax.experimental.pallas import tpu_sc as plsc

IDIM = 128
HDIM = 64
KNN = 16
NB = 2
NPTS = 8192
C1 = 64
C2 = 64
D2 = 2 * HDIM

_NW = 32
_CH = 128


def _sc_gather(table, idx2):
    rows = idx2.shape[0] * idx2.shape[1]
    width = table.shape[1]
    rows_pw = rows // _NW
    nc = rows_pw // _CH
    mesh = plsc.VectorSubcoreMesh(core_axis_name="c", subcore_axis_name="s")

    @functools.partial(
        pl.kernel,
        mesh=mesh,
        out_type=jax.ShapeDtypeStruct((rows, width), table.dtype),
        scratch_types=[
            pltpu.VMEM((nc, _CH), jnp.int32),
            pltpu.VMEM((_CH, width), table.dtype),
            pltpu.VMEM((_CH, width), table.dtype),
            pltpu.SemaphoreType.DMA,
            pltpu.SemaphoreType.DMA,
            pltpu.SemaphoreType.DMA,
            pltpu.SemaphoreType.DMA,
        ],
    )
    def gk(table_hbm, idx_hbm, out_hbm, idx_v, rows0, rows1, gs0, gs1, ss0, ss1):
        wid = lax.axis_index("s") * 2 + lax.axis_index("c")
        base = wid * rows_pw
        pltpu.sync_copy(idx_hbm.at[pl.ds(wid * nc, nc)], idx_v)

        rbuf = (rows0, rows1)
        gs = (gs0, gs1)
        ss = (ss0, ss1)

        def g_start(c, buf):
            pltpu.async_copy(table_hbm.at[idx_v.at[c]], rbuf[buf], gs[buf])

        def g_wait(buf):
            pltpu.make_async_copy(
                table_hbm.at[idx_v.at[0]], rbuf[buf], gs[buf]).wait()

        def s_start(c, buf):
            pltpu.async_copy(
                rbuf[buf], out_hbm.at[pl.ds(base + c * _CH, _CH)], ss[buf])

        def s_wait(buf):
            pltpu.make_async_copy(
                rbuf[buf], out_hbm.at[pl.ds(base, _CH)], ss[buf]).wait()

        g_start(0, 0)
        g_start(1, 1)
        g_wait(0)
        s_start(0, 0)

        def body(q, carry):
            c0 = 2 * q + 2
            s_wait(0)
            g_start(c0, 0)
            g_wait(1)
            s_start(c0 - 1, 1)
            s_wait(1)
            g_start(c0 + 1, 1)
            g_wait(0)
            s_start(c0, 0)
            return carry

        lax.fori_loop(0, (nc - 2) // 2, body, 0)
        g_wait(1)
        s_start(nc - 1, 1)
        s_wait(0)
        s_wait(1)

    return gk(table, idx2)


_T1 = 2048
_T = 256


def _p1_body(x_ref, logs_ref, bias_ref, w_ref, b1_ref, z_ref, a_ref, bm_ref):
    z = x_ref[...] * jnp.exp(logs_ref[...]) + bias_ref[...]
    z_ref[...] = z
    cat = jnp.dot(z[:, :C1], w_ref[...], preferred_element_type=jnp.float32)
    a_ref[...] = cat[:, :D2] + b1_ref[...]
    bm_ref[...] = cat[:, D2:]


def _coupling_core(g_ref, a_ref, z_ref, w2_ref, b2_ref, w3_ref, b3_ref):
    g = g_ref[...]
    a = a_ref[...]
    h1 = jax.nn.relu(g.reshape(_T, KNN, D2) + a[:, None, :]).reshape(_T * KNN, D2)
    h2 = jax.nn.relu(
        jnp.dot(h1, w2_ref[...], preferred_element_type=jnp.float32) + b2_ref[...])
    m = jnp.max(h2.reshape(_T, KNN, D2), axis=1)
    o = jnp.dot(m, w3_ref[...], preferred_element_type=jnp.float32) + b3_ref[...]
    sl = o[:, :C2] + 2.0
    s = jax.nn.sigmoid(sl)
    z = z_ref[...]
    xb2 = (z[:, C1:] + o[:, C2:]) * s
    z1 = jnp.concatenate([z[:, :C1], xb2], axis=1)
    part = -jnp.sum(jnp.maximum(-sl, 0.0) + jnp.log1p(jnp.exp(-jnp.abs(sl))))
    return z1, part


def _p3_body(g_ref, a_ref, z_ref, logs_ref, w2_ref, b2_ref, w3_ref, b3_ref,
             p_ref, w1n_ref, b1n_ref, zp_ref, a2_ref, bm2_ref, ld_ref):
    t_idx = pl.program_id(0)
    z1, part = _coupling_core(g_ref, a_ref, z_ref, w2_ref, b2_ref, w3_ref, b3_ref)
    zp = jnp.dot(z1, p_ref[...], preferred_element_type=jnp.float32)
    zp_ref[...] = zp
    cat = jnp.dot(zp[:, :C1], w1n_ref[...], preferred_element_type=jnp.float32)
    a2_ref[...] = cat[:, :D2] + b1n_ref[...]
    bm2_ref[...] = cat[:, D2:]

    @pl.when(t_idx == 0)
    def _():
        ld_ref[0, 0] = jnp.sum(logs_ref[...]) * NPTS + part

    @pl.when(t_idx != 0)
    def _():
        ld_ref[0, 0] += part


def _p5_body(g_ref, a_ref, z_ref, ld3_ref, w2_ref, b2_ref, w3_ref, b3_ref,
             zf_ref, ld_ref):
    t_idx = pl.program_id(0)
    z1, part = _coupling_core(g_ref, a_ref, z_ref, w2_ref, b2_ref, w3_ref, b3_ref)
    zf_ref[...] = z1

    @pl.when(t_idx == 0)
    def _():
        ld_ref[0, 0] = ld3_ref[0, 0] + part

    @pl.when(t_idx != 0)
    def _():
        ld_ref[0, 0] += part


def _fold_l1(p):
    w = p["l1"]["w"]
    wc, wd, wn = w[:C1], w[C1:2 * C1], w[2 * C1:]
    return wc - wd, wd + wn, p["l1"]["b"]


def _pack_coupling(pF, pG):
    waF, wbF, b1F = _fold_l1(pF)
    waG, wbG, b1G = _fold_l1(pG)
    w1 = jnp.concatenate([waF, waG, wbF, wbG], axis=1)
    b1 = jnp.concatenate([b1F, b1G])[None, :]
    w2 = jnp.zeros((D2, D2), jnp.float32)
    w2 = w2.at[:HDIM, :HDIM].set(pF["l2"]["w"]).at[HDIM:, HDIM:].set(pG["l2"]["w"])
    b2 = jnp.concatenate([pF["l2"]["b"], pG["l2"]["b"]])[None, :]
    w3 = jnp.zeros((D2, 2 * C2), jnp.float32)
    w3 = w3.at[:HDIM, :C2].set(pF["l3"]["w"]).at[HDIM:, C2:].set(pG["l3"]["w"])
    b3 = jnp.concatenate([pF["l3"]["b"], pG["l3"]["b"]])[None, :]
    return w1, b1, w2, b2, w3, b3


_SEM_ARB = pltpu.CompilerParams(dimension_semantics=("arbitrary",))


def _full2d(r, c):
    return pl.BlockSpec((r, c), lambda t: (0, 0))


def kernel(x, knn_idx, params, perm):
    x = x.astype(jnp.float32)
    idx2 = knn_idx.astype(jnp.int32).reshape(NB, (NPTS * KNN) // _CH, _CH)
    logs = params["actnorm"]["logs"].reshape(1, IDIM)
    bias = params["actnorm"]["bias"].reshape(1, IDIM)
    w1c1, b1c1, w2c1, b2c1, w3c1, b3c1 = _pack_coupling(params["c1F"], params["c1G"])
    w1c2, b1c2, w2c2, b2c2, w3c2, b3c2 = _pack_coupling(params["c2F"], params["c2G"])
    pmat = (jnp.arange(IDIM)[:, None] == perm[None, :].astype(jnp.int32)
            ).astype(jnp.float32)

    f32 = jnp.float32
    nt1 = NPTS // _T1
    nt = NPTS // _T

    p1 = pl.pallas_call(
        _p1_body,
        grid=(nt1,),
        in_specs=[
            pl.BlockSpec((_T1, IDIM), lambda t: (t, 0)),
            _full2d(1, IDIM), _full2d(1, IDIM),
            _full2d(C1, 2 * D2), _full2d(1, D2),
        ],
        out_specs=[
            pl.BlockSpec((_T1, IDIM), lambda t: (t, 0)),
            pl.BlockSpec((_T1, D2), lambda t: (t, 0)),
            pl.BlockSpec((_T1, D2), lambda t: (t, 0)),
        ],
        out_shape=[
            jax.ShapeDtypeStruct((NPTS, IDIM), f32),
            jax.ShapeDtypeStruct((NPTS, D2), f32),
            jax.ShapeDtypeStruct((NPTS, D2), f32),
        ],
        compiler_params=_SEM_ARB,
    )

    p3 = pl.pallas_call(
        _p3_body,
        grid=(nt,),
        in_specs=[
            pl.BlockSpec((_T * KNN, D2), lambda t: (t, 0)),
            pl.BlockSpec((_T, D2), lambda t: (t, 0)),
            pl.BlockSpec((_T, IDIM), lambda t: (t, 0)),
            _full2d(1, IDIM),
            _full2d(D2, D2), _full2d(1, D2),
            _full2d(D2, 2 * C2), _full2d(1, 2 * C2),
            _full2d(IDIM, IDIM),
            _full2d(C1, 2 * D2), _full2d(1, D2),
        ],
        out_specs=[
            pl.BlockSpec((_T, IDIM), lambda t: (t, 0)),
            pl.BlockSpec((_T, D2), lambda t: (t, 0)),
            pl.BlockSpec((_T, D2), lambda t: (t, 0)),
            pl.BlockSpec((1, 1), lambda t: (0, 0), memory_space=pltpu.SMEM),
        ],
        out_shape=[
            jax.ShapeDtypeStruct((NPTS, IDIM), f32),
            jax.ShapeDtypeStruct((NPTS, D2), f32),
            jax.ShapeDtypeStruct((NPTS, D2), f32),
            jax.ShapeDtypeStruct((1, 1), f32),
        ],
        compiler_params=_SEM_ARB,
    )

    p5 = pl.pallas_call(
        _p5_body,
        grid=(nt,),
        in_specs=[
            pl.BlockSpec((_T * KNN, D2), lambda t: (t, 0)),
            pl.BlockSpec((_T, D2), lambda t: (t, 0)),
            pl.BlockSpec((_T, IDIM), lambda t: (t, 0)),
            pl.BlockSpec((1, 1), lambda t: (0, 0), memory_space=pltpu.SMEM),
            _full2d(D2, D2), _full2d(1, D2),
            _full2d(D2, 2 * C2), _full2d(1, 2 * C2),
        ],
        out_specs=[
            pl.BlockSpec((_T, IDIM), lambda t: (t, 0)),
            pl.BlockSpec((1, 1), lambda t: (0, 0), memory_space=pltpu.SMEM),
        ],
        out_shape=[
            jax.ShapeDtypeStruct((NPTS, IDIM), f32),
            jax.ShapeDtypeStruct((1, 1), f32),
        ],
        compiler_params=_SEM_ARB,
    )

    zf_all, ld_all = [], []
    for b in range(NB):
        z0, a1, bm1 = p1(x[b], logs, bias, w1c1, b1c1)
        g1 = _sc_gather(bm1, idx2[b])
        zp, a2, bm2, ld3 = p3(g1, a1, z0, logs, w2c1, b2c1, w3c1, b3c1,
                              pmat, w1c2, b1c2)
        g2 = _sc_gather(bm2, idx2[b])
        zf, ld = p5(g2, a2, zp, ld3, w2c2, b2c2, w3c2, b3c2)
        zf_all.append(zf)
        ld_all.append(ld[0, 0])

    return jnp.stack(zf_all), jnp.stack(ld_all)


# speedup vs baseline: 1.9149x; 1.0552x over previous
import functools

import jax
import jax.numpy as jnp
from jax import lax
from jax.experimental import pallas as pl
from jax.experimental.pallas import tpu as pltpu
from jax.experimental.pallas import tpu_sc as plsc

IDIM = 128
HDIM = 64
KNN = 16
NB = 2
NPTS = 8192
C1 = 64
C2 = 64
D2 = 2 * HDIM

_NW = 32
_CH = 128


def _sc_gather(table, idx2):
    rows = idx2.shape[0] * idx2.shape[1]
    width = table.shape[1]
    rows_pw = rows // _NW
    nc = rows_pw // _CH
    mesh = plsc.VectorSubcoreMesh(core_axis_name="c", subcore_axis_name="s")

    @functools.partial(
        pl.kernel,
        mesh=mesh,
        out_type=jax.ShapeDtypeStruct((rows, width), table.dtype),
        scratch_types=[
            pltpu.VMEM((nc, _CH), jnp.int32),
            pltpu.VMEM((_CH, width), table.dtype),
            pltpu.VMEM((_CH, width), table.dtype),
            pltpu.SemaphoreType.DMA,
            pltpu.SemaphoreType.DMA,
            pltpu.SemaphoreType.DMA,
            pltpu.SemaphoreType.DMA,
        ],
    )
    def gk(table_hbm, idx_hbm, out_hbm, idx_v, rows0, rows1, gs0, gs1, ss0, ss1):
        wid = lax.axis_index("s") * 2 + lax.axis_index("c")
        base = wid * rows_pw
        pltpu.sync_copy(idx_hbm.at[pl.ds(wid * nc, nc)], idx_v)

        rbuf = (rows0, rows1)
        gs = (gs0, gs1)
        ss = (ss0, ss1)

        def g_start(c, buf):
            pltpu.async_copy(table_hbm.at[idx_v.at[c]], rbuf[buf], gs[buf])

        def g_wait(buf):
            pltpu.make_async_copy(
                table_hbm.at[idx_v.at[0]], rbuf[buf], gs[buf]).wait()

        def s_start(c, buf):
            pltpu.async_copy(
                rbuf[buf], out_hbm.at[pl.ds(base + c * _CH, _CH)], ss[buf])

        def s_wait(buf):
            pltpu.make_async_copy(
                rbuf[buf], out_hbm.at[pl.ds(base, _CH)], ss[buf]).wait()

        g_start(0, 0)
        g_start(1, 1)
        g_wait(0)
        s_start(0, 0)

        def body(q, carry):
            c0 = 2 * q + 2
            s_wait(0)
            g_start(c0, 0)
            g_wait(1)
            s_start(c0 - 1, 1)
            s_wait(1)
            g_start(c0 + 1, 1)
            g_wait(0)
            s_start(c0, 0)
            return carry

        lax.fori_loop(0, (nc - 2) // 2, body, 0)
        g_wait(1)
        s_start(nc - 1, 1)
        s_wait(0)
        s_wait(1)

    return gk(table, idx2)


_T1 = 2048
_T = 512


def _p1_body(x_ref, logs_ref, bias_ref, w_ref, b1_ref, z_ref, a_ref, bm_ref):
    z = x_ref[...] * jnp.exp(logs_ref[...]) + bias_ref[...]
    z_ref[...] = z
    cat = jnp.dot(z[:, :C1], w_ref[...], preferred_element_type=jnp.float32)
    a_ref[...] = cat[:, :D2] + b1_ref[...]
    bm_ref[...] = cat[:, D2:]


def _coupling_core(g_ref, a_ref, z_ref, w2_ref, b2_ref, w3_ref, b3_ref):
    g = g_ref[...]
    a = a_ref[...]
    h1 = jax.nn.relu(g.reshape(_T, KNN, D2) + a[:, None, :]).reshape(_T * KNN, D2)
    h2 = jax.nn.relu(
        jnp.dot(h1, w2_ref[...], preferred_element_type=jnp.float32) + b2_ref[...])
    m = jnp.max(h2.reshape(_T, KNN, D2), axis=1)
    o = jnp.dot(m, w3_ref[...], preferred_element_type=jnp.float32) + b3_ref[...]
    sl = o[:, :C2] + 2.0
    s = jax.nn.sigmoid(sl)
    z = z_ref[...]
    xb2 = (z[:, C1:] + o[:, C2:]) * s
    z1 = jnp.concatenate([z[:, :C1], xb2], axis=1)
    part = -jnp.sum(jnp.maximum(-sl, 0.0) + jnp.log1p(jnp.exp(-jnp.abs(sl))))
    return z1, part


def _p3_body(g_ref, a_ref, z_ref, logs_ref, w2_ref, b2_ref, w3_ref, b3_ref,
             p_ref, w1n_ref, b1n_ref, zp_ref, a2_ref, bm2_ref, ld_ref):
    t_idx = pl.program_id(0)
    z1, part = _coupling_core(g_ref, a_ref, z_ref, w2_ref, b2_ref, w3_ref, b3_ref)
    zp = jnp.dot(z1, p_ref[...], preferred_element_type=jnp.float32)
    zp_ref[...] = zp
    cat = jnp.dot(zp[:, :C1], w1n_ref[...], preferred_element_type=jnp.float32)
    a2_ref[...] = cat[:, :D2] + b1n_ref[...]
    bm2_ref[...] = cat[:, D2:]

    @pl.when(t_idx == 0)
    def _():
        ld_ref[0, 0] = jnp.sum(logs_ref[...]) * NPTS + part

    @pl.when(t_idx != 0)
    def _():
        ld_ref[0, 0] += part


def _p5_body(g_ref, a_ref, z_ref, ld3_ref, w2_ref, b2_ref, w3_ref, b3_ref,
             zf_ref, ld_ref):
    t_idx = pl.program_id(0)
    z1, part = _coupling_core(g_ref, a_ref, z_ref, w2_ref, b2_ref, w3_ref, b3_ref)
    zf_ref[...] = z1

    @pl.when(t_idx == 0)
    def _():
        ld_ref[0, 0] = ld3_ref[0, 0] + part

    @pl.when(t_idx != 0)
    def _():
        ld_ref[0, 0] += part


def _fold_l1(p):
    w = p["l1"]["w"]
    wc, wd, wn = w[:C1], w[C1:2 * C1], w[2 * C1:]
    return wc - wd, wd + wn, p["l1"]["b"]


def _pack_coupling(pF, pG):
    waF, wbF, b1F = _fold_l1(pF)
    waG, wbG, b1G = _fold_l1(pG)
    w1 = jnp.concatenate([waF, waG, wbF, wbG], axis=1)
    b1 = jnp.concatenate([b1F, b1G])[None, :]
    w2 = jnp.zeros((D2, D2), jnp.float32)
    w2 = w2.at[:HDIM, :HDIM].set(pF["l2"]["w"]).at[HDIM:, HDIM:].set(pG["l2"]["w"])
    b2 = jnp.concatenate([pF["l2"]["b"], pG["l2"]["b"]])[None, :]
    w3 = jnp.zeros((D2, 2 * C2), jnp.float32)
    w3 = w3.at[:HDIM, :C2].set(pF["l3"]["w"]).at[HDIM:, C2:].set(pG["l3"]["w"])
    b3 = jnp.concatenate([pF["l3"]["b"], pG["l3"]["b"]])[None, :]
    return w1, b1, w2, b2, w3, b3


_SEM_ARB = pltpu.CompilerParams(dimension_semantics=("arbitrary",))


def _full2d(r, c):
    return pl.BlockSpec((r, c), lambda t: (0, 0))


def kernel(x, knn_idx, params, perm):
    x = x.astype(jnp.float32)
    idx2 = knn_idx.astype(jnp.int32).reshape(NB, (NPTS * KNN) // _CH, _CH)
    logs = params["actnorm"]["logs"].reshape(1, IDIM)
    bias = params["actnorm"]["bias"].reshape(1, IDIM)
    w1c1, b1c1, w2c1, b2c1, w3c1, b3c1 = _pack_coupling(params["c1F"], params["c1G"])
    w1c2, b1c2, w2c2, b2c2, w3c2, b3c2 = _pack_coupling(params["c2F"], params["c2G"])
    pmat = (jnp.arange(IDIM)[:, None] == perm[None, :].astype(jnp.int32)
            ).astype(jnp.float32)

    f32 = jnp.float32
    nt1 = NPTS // _T1
    nt = NPTS // _T

    p1 = pl.pallas_call(
        _p1_body,
        grid=(nt1,),
        in_specs=[
            pl.BlockSpec((_T1, IDIM), lambda t: (t, 0)),
            _full2d(1, IDIM), _full2d(1, IDIM),
            _full2d(C1, 2 * D2), _full2d(1, D2),
        ],
        out_specs=[
            pl.BlockSpec((_T1, IDIM), lambda t: (t, 0)),
            pl.BlockSpec((_T1, D2), lambda t: (t, 0)),
            pl.BlockSpec((_T1, D2), lambda t: (t, 0)),
        ],
        out_shape=[
            jax.ShapeDtypeStruct((NPTS, IDIM), f32),
            jax.ShapeDtypeStruct((NPTS, D2), f32),
            jax.ShapeDtypeStruct((NPTS, D2), f32),
        ],
        compiler_params=_SEM_ARB,
    )

    p3 = pl.pallas_call(
        _p3_body,
        grid=(nt,),
        in_specs=[
            pl.BlockSpec((_T * KNN, D2), lambda t: (t, 0)),
            pl.BlockSpec((_T, D2), lambda t: (t, 0)),
            pl.BlockSpec((_T, IDIM), lambda t: (t, 0)),
            _full2d(1, IDIM),
            _full2d(D2, D2), _full2d(1, D2),
            _full2d(D2, 2 * C2), _full2d(1, 2 * C2),
            _full2d(IDIM, IDIM),
            _full2d(C1, 2 * D2), _full2d(1, D2),
        ],
        out_specs=[
            pl.BlockSpec((_T, IDIM), lambda t: (t, 0)),
            pl.BlockSpec((_T, D2), lambda t: (t, 0)),
            pl.BlockSpec((_T, D2), lambda t: (t, 0)),
            pl.BlockSpec((1, 1), lambda t: (0, 0), memory_space=pltpu.SMEM),
        ],
        out_shape=[
            jax.ShapeDtypeStruct((NPTS, IDIM), f32),
            jax.ShapeDtypeStruct((NPTS, D2), f32),
            jax.ShapeDtypeStruct((NPTS, D2), f32),
            jax.ShapeDtypeStruct((1, 1), f32),
        ],
        compiler_params=_SEM_ARB,
    )

    p5 = pl.pallas_call(
        _p5_body,
        grid=(nt,),
        in_specs=[
            pl.BlockSpec((_T * KNN, D2), lambda t: (t, 0)),
            pl.BlockSpec((_T, D2), lambda t: (t, 0)),
            pl.BlockSpec((_T, IDIM), lambda t: (t, 0)),
            pl.BlockSpec((1, 1), lambda t: (0, 0), memory_space=pltpu.SMEM),
            _full2d(D2, D2), _full2d(1, D2),
            _full2d(D2, 2 * C2), _full2d(1, 2 * C2),
        ],
        out_specs=[
            pl.BlockSpec((_T, IDIM), lambda t: (t, 0)),
            pl.BlockSpec((1, 1), lambda t: (0, 0), memory_space=pltpu.SMEM),
        ],
        out_shape=[
            jax.ShapeDtypeStruct((NPTS, IDIM), f32),
            jax.ShapeDtypeStruct((1, 1), f32),
        ],
        compiler_params=_SEM_ARB,
    )

    st1 = [p1(x[b], logs, bias, w1c1, b1c1) for b in range(NB)]
    g1 = [_sc_gather(st1[b][2], idx2[b]) for b in range(NB)]
    st3 = [p3(g1[b], st1[b][1], st1[b][0], logs, w2c1, b2c1, w3c1, b3c1,
              pmat, w1c2, b1c2) for b in range(NB)]
    g2 = [_sc_gather(st3[b][2], idx2[b]) for b in range(NB)]
    st5 = [p5(g2[b], st3[b][1], st3[b][0], st3[b][3], w2c2, b2c2, w3c2, b3c2)
           for b in range(NB)]

    return (jnp.stack([st5[b][0] for b in range(NB)]),
            jnp.stack([st5[b][1][0, 0] for b in range(NB)]))


# speedup vs baseline: 2.5175x; 1.3147x over previous
import functools

import jax
import jax.numpy as jnp
from jax import lax
from jax.experimental import pallas as pl
from jax.experimental.pallas import tpu as pltpu
from jax.experimental.pallas import tpu_sc as plsc

IDIM = 128
HDIM = 64
KNN = 16
NB = 2
NPTS = 8192
C1 = 64
C2 = 64
D2 = 2 * HDIM

_NW = 32
_CH = 128


def _sc_gather(table, idx2):
    rows = idx2.shape[0] * idx2.shape[1]
    width = table.shape[1]
    rows_pw = rows // _NW
    nc = rows_pw // _CH
    mesh = plsc.VectorSubcoreMesh(core_axis_name="c", subcore_axis_name="s")

    @functools.partial(
        pl.kernel,
        mesh=mesh,
        out_type=jax.ShapeDtypeStruct((rows, width), table.dtype),
        scratch_types=[
            pltpu.VMEM_SHARED((NPTS, IDIM), jnp.float32),
            pltpu.VMEM((nc, _CH), jnp.int32),
            pltpu.VMEM((_CH, width), table.dtype),
            pltpu.VMEM((_CH, width), table.dtype),
            pltpu.SemaphoreType.DMA,
            pltpu.SemaphoreType.DMA,
            pltpu.SemaphoreType.DMA,
            pltpu.SemaphoreType.DMA,
        ],
    )
    def gk(table_hbm, idx_hbm, out_hbm, tshr, idx_v, rows0, rows1,
           gs0, gs1, ss0, ss1):
        wid = lax.axis_index("s") * 2 + lax.axis_index("c")
        base = wid * rows_pw

        @pl.when(lax.axis_index("s") == 0)
        def _():
            pltpu.sync_copy(table_hbm, tshr)

        plsc.subcore_barrier()
        pltpu.sync_copy(idx_hbm.at[pl.ds(wid * nc, nc)], idx_v)

        rbuf = (rows0, rows1)
        gs = (gs0, gs1)
        ss = (ss0, ss1)

        def g_start(c, buf):
            pltpu.async_copy(tshr.at[idx_v.at[c]], rbuf[buf], gs[buf])

        def g_wait(buf):
            pltpu.make_async_copy(
                tshr.at[idx_v.at[0]], rbuf[buf], gs[buf]).wait()

        def s_start(c, buf):
            pltpu.async_copy(
                rbuf[buf], out_hbm.at[pl.ds(base + c * _CH, _CH)], ss[buf])

        def s_wait(buf):
            pltpu.make_async_copy(
                rbuf[buf], out_hbm.at[pl.ds(base, _CH)], ss[buf]).wait()

        g_start(0, 0)
        g_start(1, 1)
        g_wait(0)
        s_start(0, 0)

        def body(q, carry):
            c0 = 2 * q + 2
            s_wait(0)
            g_start(c0, 0)
            g_wait(1)
            s_start(c0 - 1, 1)
            s_wait(1)
            g_start(c0 + 1, 1)
            g_wait(0)
            s_start(c0, 0)
            return carry

        lax.fori_loop(0, (nc - 2) // 2, body, 0)
        g_wait(1)
        s_start(nc - 1, 1)
        s_wait(0)
        s_wait(1)

    return gk(table, idx2)


_T1 = 2048
_T = 512


def _p1_body(x_ref, logs_ref, bias_ref, w_ref, b1_ref, z_ref, a_ref, bm_ref):
    z = x_ref[...] * jnp.exp(logs_ref[...]) + bias_ref[...]
    z_ref[...] = z
    cat = jnp.dot(z[:, :C1], w_ref[...], preferred_element_type=jnp.float32)
    a_ref[...] = cat[:, :D2] + b1_ref[...]
    bm_ref[...] = cat[:, D2:]


def _coupling_core(g_ref, a_ref, z_ref, w2_ref, b2_ref, w3_ref, b3_ref):
    g = g_ref[...]
    a = a_ref[...]
    h1 = jax.nn.relu(g.reshape(_T, KNN, D2) + a[:, None, :]).reshape(_T * KNN, D2)
    h2 = jax.nn.relu(
        jnp.dot(h1, w2_ref[...], preferred_element_type=jnp.float32) + b2_ref[...])
    m = jnp.max(h2.reshape(_T, KNN, D2), axis=1)
    o = jnp.dot(m, w3_ref[...], preferred_element_type=jnp.float32) + b3_ref[...]
    sl = o[:, :C2] + 2.0
    s = jax.nn.sigmoid(sl)
    z = z_ref[...]
    xb2 = (z[:, C1:] + o[:, C2:]) * s
    z1 = jnp.concatenate([z[:, :C1], xb2], axis=1)
    part = -jnp.sum(jnp.maximum(-sl, 0.0) + jnp.log1p(jnp.exp(-jnp.abs(sl))))
    return z1, part


def _p3_body(g_ref, a_ref, z_ref, logs_ref, w2_ref, b2_ref, w3_ref, b3_ref,
             p_ref, w1n_ref, b1n_ref, zp_ref, a2_ref, bm2_ref, ld_ref):
    t_idx = pl.program_id(0)
    z1, part = _coupling_core(g_ref, a_ref, z_ref, w2_ref, b2_ref, w3_ref, b3_ref)
    zp = jnp.dot(z1, p_ref[...], preferred_element_type=jnp.float32)
    zp_ref[...] = zp
    cat = jnp.dot(zp[:, :C1], w1n_ref[...], preferred_element_type=jnp.float32)
    a2_ref[...] = cat[:, :D2] + b1n_ref[...]
    bm2_ref[...] = cat[:, D2:]

    @pl.when(t_idx == 0)
    def _():
        ld_ref[0, 0] = jnp.sum(logs_ref[...]) * NPTS + part

    @pl.when(t_idx != 0)
    def _():
        ld_ref[0, 0] += part


def _p5_body(g_ref, a_ref, z_ref, ld3_ref, w2_ref, b2_ref, w3_ref, b3_ref,
             zf_ref, ld_ref):
    t_idx = pl.program_id(0)
    z1, part = _coupling_core(g_ref, a_ref, z_ref, w2_ref, b2_ref, w3_ref, b3_ref)
    zf_ref[...] = z1

    @pl.when(t_idx == 0)
    def _():
        ld_ref[0, 0] = ld3_ref[0, 0] + part

    @pl.when(t_idx != 0)
    def _():
        ld_ref[0, 0] += part


def _fold_l1(p):
    w = p["l1"]["w"]
    wc, wd, wn = w[:C1], w[C1:2 * C1], w[2 * C1:]
    return wc - wd, wd + wn, p["l1"]["b"]


def _pack_coupling(pF, pG):
    waF, wbF, b1F = _fold_l1(pF)
    waG, wbG, b1G = _fold_l1(pG)
    w1 = jnp.concatenate([waF, waG, wbF, wbG], axis=1)
    b1 = jnp.concatenate([b1F, b1G])[None, :]
    w2 = jnp.zeros((D2, D2), jnp.float32)
    w2 = w2.at[:HDIM, :HDIM].set(pF["l2"]["w"]).at[HDIM:, HDIM:].set(pG["l2"]["w"])
    b2 = jnp.concatenate([pF["l2"]["b"], pG["l2"]["b"]])[None, :]
    w3 = jnp.zeros((D2, 2 * C2), jnp.float32)
    w3 = w3.at[:HDIM, :C2].set(pF["l3"]["w"]).at[HDIM:, C2:].set(pG["l3"]["w"])
    b3 = jnp.concatenate([pF["l3"]["b"], pG["l3"]["b"]])[None, :]
    return w1, b1, w2, b2, w3, b3


_SEM_ARB = pltpu.CompilerParams(dimension_semantics=("arbitrary",))


def _full2d(r, c):
    return pl.BlockSpec((r, c), lambda t: (0, 0))


def kernel(x, knn_idx, params, perm):
    x = x.astype(jnp.float32)
    idx2 = knn_idx.astype(jnp.int32).reshape(NB, (NPTS * KNN) // _CH, _CH)
    logs = params["actnorm"]["logs"].reshape(1, IDIM)
    bias = params["actnorm"]["bias"].reshape(1, IDIM)
    w1c1, b1c1, w2c1, b2c1, w3c1, b3c1 = _pack_coupling(params["c1F"], params["c1G"])
    w1c2, b1c2, w2c2, b2c2, w3c2, b3c2 = _pack_coupling(params["c2F"], params["c2G"])
    pmat = (jnp.arange(IDIM)[:, None] == perm[None, :].astype(jnp.int32)
            ).astype(jnp.float32)

    f32 = jnp.float32
    nt1 = NPTS // _T1
    nt = NPTS // _T

    p1 = pl.pallas_call(
        _p1_body,
        grid=(nt1,),
        in_specs=[
            pl.BlockSpec((_T1, IDIM), lambda t: (t, 0)),
            _full2d(1, IDIM), _full2d(1, IDIM),
            _full2d(C1, 2 * D2), _full2d(1, D2),
        ],
        out_specs=[
            pl.BlockSpec((_T1, IDIM), lambda t: (t, 0)),
            pl.BlockSpec((_T1, D2), lambda t: (t, 0)),
            pl.BlockSpec((_T1, D2), lambda t: (t, 0)),
        ],
        out_shape=[
            jax.ShapeDtypeStruct((NPTS, IDIM), f32),
            jax.ShapeDtypeStruct((NPTS, D2), f32),
            jax.ShapeDtypeStruct((NPTS, D2), f32),
        ],
        compiler_params=_SEM_ARB,
    )

    p3 = pl.pallas_call(
        _p3_body,
        grid=(nt,),
        in_specs=[
            pl.BlockSpec((_T * KNN, D2), lambda t: (t, 0)),
            pl.BlockSpec((_T, D2), lambda t: (t, 0)),
            pl.BlockSpec((_T, IDIM), lambda t: (t, 0)),
            _full2d(1, IDIM),
            _full2d(D2, D2), _full2d(1, D2),
            _full2d(D2, 2 * C2), _full2d(1, 2 * C2),
            _full2d(IDIM, IDIM),
            _full2d(C1, 2 * D2), _full2d(1, D2),
        ],
        out_specs=[
            pl.BlockSpec((_T, IDIM), lambda t: (t, 0)),
            pl.BlockSpec((_T, D2), lambda t: (t, 0)),
            pl.BlockSpec((_T, D2), lambda t: (t, 0)),
            pl.BlockSpec((1, 1), lambda t: (0, 0), memory_space=pltpu.SMEM),
        ],
        out_shape=[
            jax.ShapeDtypeStruct((NPTS, IDIM), f32),
            jax.ShapeDtypeStruct((NPTS, D2), f32),
            jax.ShapeDtypeStruct((NPTS, D2), f32),
            jax.ShapeDtypeStruct((1, 1), f32),
        ],
        compiler_params=_SEM_ARB,
    )

    p5 = pl.pallas_call(
        _p5_body,
        grid=(nt,),
        in_specs=[
            pl.BlockSpec((_T * KNN, D2), lambda t: (t, 0)),
            pl.BlockSpec((_T, D2), lambda t: (t, 0)),
            pl.BlockSpec((_T, IDIM), lambda t: (t, 0)),
            pl.BlockSpec((1, 1), lambda t: (0, 0), memory_space=pltpu.SMEM),
            _full2d(D2, D2), _full2d(1, D2),
            _full2d(D2, 2 * C2), _full2d(1, 2 * C2),
        ],
        out_specs=[
            pl.BlockSpec((_T, IDIM), lambda t: (t, 0)),
            pl.BlockSpec((1, 1), lambda t: (0, 0), memory_space=pltpu.SMEM),
        ],
        out_shape=[
            jax.ShapeDtypeStruct((NPTS, IDIM), f32),
            jax.ShapeDtypeStruct((1, 1), f32),
        ],
        compiler_params=_SEM_ARB,
    )

    st1 = [p1(x[b], logs, bias, w1c1, b1c1) for b in range(NB)]
    g1 = [_sc_gather(st1[b][2], idx2[b]) for b in range(NB)]
    st3 = [p3(g1[b], st1[b][1], st1[b][0], logs, w2c1, b2c1, w3c1, b3c1,
              pmat, w1c2, b1c2) for b in range(NB)]
    g2 = [_sc_gather(st3[b][2], idx2[b]) for b in range(NB)]
    st5 = [p5(g2[b], st3[b][1], st3[b][0], st3[b][3], w2c2, b2c2, w3c2, b3c2)
           for b in range(NB)]

    return (jnp.stack([st5[b][0] for b in range(NB)]),
            jnp.stack([st5[b][1][0, 0] for b in range(NB)]))


# speedup vs baseline: 2.5180x; 1.0002x over previous
import functools

import jax
import jax.numpy as jnp
from jax import lax
from jax.experimental import pallas as pl
from jax.experimental.pallas import tpu as pltpu
from jax.experimental.pallas import tpu_sc as plsc

IDIM = 128
HDIM = 64
KNN = 16
NB = 2
NPTS = 8192
C1 = 64
C2 = 64
D2 = 2 * HDIM

_NW = 32
_CH = 128


def _sc_gather(table, idx2):
    rows = idx2.shape[0] * idx2.shape[1]
    width = table.shape[1]
    rows_pw = rows // _NW
    nc = rows_pw // _CH
    mesh = plsc.VectorSubcoreMesh(core_axis_name="c", subcore_axis_name="s")

    @functools.partial(
        pl.kernel,
        mesh=mesh,
        out_type=jax.ShapeDtypeStruct((rows, width), table.dtype),
        scratch_types=[
            pltpu.VMEM_SHARED((NPTS, IDIM), jnp.float32),
            pltpu.VMEM((nc, _CH), jnp.int32),
            pltpu.VMEM((_CH, width), table.dtype),
            pltpu.VMEM((_CH, width), table.dtype),
            pltpu.VMEM((_CH, width), table.dtype),
            pltpu.SemaphoreType.DMA,
            pltpu.SemaphoreType.DMA,
            pltpu.SemaphoreType.DMA,
            pltpu.SemaphoreType.DMA,
            pltpu.SemaphoreType.DMA,
            pltpu.SemaphoreType.DMA,
        ],
    )
    def gk(table_hbm, idx_hbm, out_hbm, tshr, idx_v, r0, r1, r2,
           g0, g1, g2, s0, s1, s2):
        wid = lax.axis_index("s") * 2 + lax.axis_index("c")
        base = wid * rows_pw

        @pl.when(lax.axis_index("s") == 0)
        def _():
            pltpu.sync_copy(table_hbm, tshr)

        plsc.subcore_barrier()
        pltpu.sync_copy(idx_hbm.at[pl.ds(wid * nc, nc)], idx_v)

        nbuf = 3
        rbuf = (r0, r1, r2)
        gs = (g0, g1, g2)
        ss = (s0, s1, s2)

        def g_start(c, buf):
            pltpu.async_copy(tshr.at[idx_v.at[c]], rbuf[buf], gs[buf])

        def g_wait(buf):
            pltpu.make_async_copy(
                tshr.at[idx_v.at[0]], rbuf[buf], gs[buf]).wait()

        def s_start(c, buf):
            pltpu.async_copy(
                rbuf[buf], out_hbm.at[pl.ds(base + c * _CH, _CH)], ss[buf])

        def s_wait(buf):
            pltpu.make_async_copy(
                rbuf[buf], out_hbm.at[pl.ds(base, _CH)], ss[buf]).wait()

        for b in range(nbuf):
            g_start(b, b)
        for c in range(nc):
            buf = c % nbuf
            g_wait(buf)
            s_start(c, buf)
            if c + nbuf < nc:
                s_wait(buf)
                g_start(c + nbuf, buf)
        for c in range(max(nc - nbuf, 0), nc):
            s_wait(c % nbuf)

    return gk(table, idx2)


_T1 = 2048
_T = 512


def _p1_body(x_ref, logs_ref, bias_ref, w_ref, b1_ref, z_ref, a_ref, bm_ref):
    z = x_ref[...] * jnp.exp(logs_ref[...]) + bias_ref[...]
    z_ref[...] = z
    cat = jnp.dot(z[:, :C1], w_ref[...], preferred_element_type=jnp.float32)
    a_ref[...] = cat[:, :D2] + b1_ref[...]
    bm_ref[...] = cat[:, D2:]


def _coupling_core(g_ref, a_ref, z_ref, w2_ref, b2_ref, w3_ref, b3_ref):
    g = g_ref[...]
    a = a_ref[...]
    h1 = jax.nn.relu(g.reshape(_T, KNN, D2) + a[:, None, :]).reshape(_T * KNN, D2)
    h2 = jax.nn.relu(
        jnp.dot(h1, w2_ref[...], preferred_element_type=jnp.float32) + b2_ref[...])
    m = jnp.max(h2.reshape(_T, KNN, D2), axis=1)
    o = jnp.dot(m, w3_ref[...], preferred_element_type=jnp.float32) + b3_ref[...]
    sl = o[:, :C2] + 2.0
    s = jax.nn.sigmoid(sl)
    z = z_ref[...]
    xb2 = (z[:, C1:] + o[:, C2:]) * s
    z1 = jnp.concatenate([z[:, :C1], xb2], axis=1)
    part = -jnp.sum(jnp.maximum(-sl, 0.0) + jnp.log1p(jnp.exp(-jnp.abs(sl))))
    return z1, part


def _p3_body(g_ref, a_ref, z_ref, logs_ref, w2_ref, b2_ref, w3_ref, b3_ref,
             p_ref, w1n_ref, b1n_ref, zp_ref, a2_ref, bm2_ref, ld_ref):
    t_idx = pl.program_id(0)
    z1, part = _coupling_core(g_ref, a_ref, z_ref, w2_ref, b2_ref, w3_ref, b3_ref)
    zp = jnp.dot(z1, p_ref[...], preferred_element_type=jnp.float32)
    zp_ref[...] = zp
    cat = jnp.dot(zp[:, :C1], w1n_ref[...], preferred_element_type=jnp.float32)
    a2_ref[...] = cat[:, :D2] + b1n_ref[...]
    bm2_ref[...] = cat[:, D2:]

    @pl.when(t_idx == 0)
    def _():
        ld_ref[0, 0] = jnp.sum(logs_ref[...]) * NPTS + part

    @pl.when(t_idx != 0)
    def _():
        ld_ref[0, 0] += part


def _p5_body(g_ref, a_ref, z_ref, ld3_ref, w2_ref, b2_ref, w3_ref, b3_ref,
             zf_ref, ld_ref):
    t_idx = pl.program_id(0)
    z1, part = _coupling_core(g_ref, a_ref, z_ref, w2_ref, b2_ref, w3_ref, b3_ref)
    zf_ref[...] = z1

    @pl.when(t_idx == 0)
    def _():
        ld_ref[0, 0] = ld3_ref[0, 0] + part

    @pl.when(t_idx != 0)
    def _():
        ld_ref[0, 0] += part


def _fold_l1(p):
    w = p["l1"]["w"]
    wc, wd, wn = w[:C1], w[C1:2 * C1], w[2 * C1:]
    return wc - wd, wd + wn, p["l1"]["b"]


def _pack_coupling(pF, pG):
    waF, wbF, b1F = _fold_l1(pF)
    waG, wbG, b1G = _fold_l1(pG)
    w1 = jnp.concatenate([waF, waG, wbF, wbG], axis=1)
    b1 = jnp.concatenate([b1F, b1G])[None, :]
    w2 = jnp.zeros((D2, D2), jnp.float32)
    w2 = w2.at[:HDIM, :HDIM].set(pF["l2"]["w"]).at[HDIM:, HDIM:].set(pG["l2"]["w"])
    b2 = jnp.concatenate([pF["l2"]["b"], pG["l2"]["b"]])[None, :]
    w3 = jnp.zeros((D2, 2 * C2), jnp.float32)
    w3 = w3.at[:HDIM, :C2].set(pF["l3"]["w"]).at[HDIM:, C2:].set(pG["l3"]["w"])
    b3 = jnp.concatenate([pF["l3"]["b"], pG["l3"]["b"]])[None, :]
    return w1, b1, w2, b2, w3, b3


_SEM_ARB = pltpu.CompilerParams(dimension_semantics=("arbitrary",))


def _full2d(r, c):
    return pl.BlockSpec((r, c), lambda t: (0, 0))


def kernel(x, knn_idx, params, perm):
    x = x.astype(jnp.float32)
    idx2 = knn_idx.astype(jnp.int32).reshape(NB, (NPTS * KNN) // _CH, _CH)
    logs = params["actnorm"]["logs"].reshape(1, IDIM)
    bias = params["actnorm"]["bias"].reshape(1, IDIM)
    w1c1, b1c1, w2c1, b2c1, w3c1, b3c1 = _pack_coupling(params["c1F"], params["c1G"])
    w1c2, b1c2, w2c2, b2c2, w3c2, b3c2 = _pack_coupling(params["c2F"], params["c2G"])
    pmat = (jnp.arange(IDIM)[:, None] == perm[None, :].astype(jnp.int32)
            ).astype(jnp.float32)

    f32 = jnp.float32
    nt1 = NPTS // _T1
    nt = NPTS // _T

    p1 = pl.pallas_call(
        _p1_body,
        grid=(nt1,),
        in_specs=[
            pl.BlockSpec((_T1, IDIM), lambda t: (t, 0)),
            _full2d(1, IDIM), _full2d(1, IDIM),
            _full2d(C1, 2 * D2), _full2d(1, D2),
        ],
        out_specs=[
            pl.BlockSpec((_T1, IDIM), lambda t: (t, 0)),
            pl.BlockSpec((_T1, D2), lambda t: (t, 0)),
            pl.BlockSpec((_T1, D2), lambda t: (t, 0)),
        ],
        out_shape=[
            jax.ShapeDtypeStruct((NPTS, IDIM), f32),
            jax.ShapeDtypeStruct((NPTS, D2), f32),
            jax.ShapeDtypeStruct((NPTS, D2), f32),
        ],
        compiler_params=_SEM_ARB,
    )

    p3 = pl.pallas_call(
        _p3_body,
        grid=(nt,),
        in_specs=[
            pl.BlockSpec((_T * KNN, D2), lambda t: (t, 0)),
            pl.BlockSpec((_T, D2), lambda t: (t, 0)),
            pl.BlockSpec((_T, IDIM), lambda t: (t, 0)),
            _full2d(1, IDIM),
            _full2d(D2, D2), _full2d(1, D2),
            _full2d(D2, 2 * C2), _full2d(1, 2 * C2),
            _full2d(IDIM, IDIM),
            _full2d(C1, 2 * D2), _full2d(1, D2),
        ],
        out_specs=[
            pl.BlockSpec((_T, IDIM), lambda t: (t, 0)),
            pl.BlockSpec((_T, D2), lambda t: (t, 0)),
            pl.BlockSpec((_T, D2), lambda t: (t, 0)),
            pl.BlockSpec((1, 1), lambda t: (0, 0), memory_space=pltpu.SMEM),
        ],
        out_shape=[
            jax.ShapeDtypeStruct((NPTS, IDIM), f32),
            jax.ShapeDtypeStruct((NPTS, D2), f32),
            jax.ShapeDtypeStruct((NPTS, D2), f32),
            jax.ShapeDtypeStruct((1, 1), f32),
        ],
        compiler_params=_SEM_ARB,
    )

    p5 = pl.pallas_call(
        _p5_body,
        grid=(nt,),
        in_specs=[
            pl.BlockSpec((_T * KNN, D2), lambda t: (t, 0)),
            pl.BlockSpec((_T, D2), lambda t: (t, 0)),
            pl.BlockSpec((_T, IDIM), lambda t: (t, 0)),
            pl.BlockSpec((1, 1), lambda t: (0, 0), memory_space=pltpu.SMEM),
            _full2d(D2, D2), _full2d(1, D2),
            _full2d(D2, 2 * C2), _full2d(1, 2 * C2),
        ],
        out_specs=[
            pl.BlockSpec((_T, IDIM), lambda t: (t, 0)),
            pl.BlockSpec((1, 1), lambda t: (0, 0), memory_space=pltpu.SMEM),
        ],
        out_shape=[
            jax.ShapeDtypeStruct((NPTS, IDIM), f32),
            jax.ShapeDtypeStruct((1, 1), f32),
        ],
        compiler_params=_SEM_ARB,
    )

    st1 = [p1(x[b], logs, bias, w1c1, b1c1) for b in range(NB)]
    g1 = [_sc_gather(st1[b][2], idx2[b]) for b in range(NB)]
    st3 = [p3(g1[b], st1[b][1], st1[b][0], logs, w2c1, b2c1, w3c1, b3c1,
              pmat, w1c2, b1c2) for b in range(NB)]
    g2 = [_sc_gather(st3[b][2], idx2[b]) for b in range(NB)]
    st5 = [p5(g2[b], st3[b][1], st3[b][0], st3[b][3], w2c2, b2c2, w3c2, b3c2)
           for b in range(NB)]

    return (jnp.stack([st5[b][0] for b in range(NB)]),
            jnp.stack([st5[b][1][0, 0] for b in range(NB)]))


# speedup vs baseline: 2.6494x; 1.0522x over previous
import functools

import jax
import jax.numpy as jnp
from jax import lax
from jax.experimental import pallas as pl
from jax.experimental.pallas import tpu as pltpu
from jax.experimental.pallas import tpu_sc as plsc

IDIM = 128
HDIM = 64
KNN = 16
NB = 2
NPTS = 8192
C1 = 64
C2 = 64
D2 = 2 * HDIM

_NW = 32
_CH = 128


def _sc_gather(table, idx2):
    rows = idx2.shape[0] * idx2.shape[1]
    width = table.shape[1]
    rows_pw = rows // _NW
    nc = rows_pw // _CH
    mesh = plsc.VectorSubcoreMesh(core_axis_name="c", subcore_axis_name="s")

    @functools.partial(
        pl.kernel,
        mesh=mesh,
        out_type=jax.ShapeDtypeStruct((rows, width), table.dtype),
        scratch_types=[
            pltpu.VMEM_SHARED((NPTS, IDIM), jnp.float32),
            pltpu.VMEM((nc, _CH), jnp.int32),
            pltpu.VMEM((_CH, width), table.dtype),
            pltpu.VMEM((_CH, width), table.dtype),
            pltpu.VMEM((_CH, width), table.dtype),
            pltpu.SemaphoreType.DMA,
            pltpu.SemaphoreType.DMA,
            pltpu.SemaphoreType.DMA,
            pltpu.SemaphoreType.DMA,
            pltpu.SemaphoreType.DMA,
            pltpu.SemaphoreType.DMA,
        ],
    )
    def gk(table_hbm, idx_hbm, out_hbm, tshr, idx_v, r0, r1, r2,
           g0, g1, g2, s0, s1, s2):
        wid = lax.axis_index("s") * 2 + lax.axis_index("c")
        base = wid * rows_pw

        @pl.when(lax.axis_index("s") == 0)
        def _():
            pltpu.sync_copy(table_hbm, tshr)

        plsc.subcore_barrier()
        pltpu.sync_copy(idx_hbm.at[pl.ds(wid * nc, nc)], idx_v)

        nbuf = 3
        rbuf = (r0, r1, r2)
        gs = (g0, g1, g2)
        ss = (s0, s1, s2)

        def g_start(c, buf):
            pltpu.async_copy(tshr.at[idx_v.at[c]], rbuf[buf], gs[buf])

        def g_wait(buf):
            pltpu.make_async_copy(
                tshr.at[idx_v.at[0]], rbuf[buf], gs[buf]).wait()

        def s_start(c, buf):
            pltpu.async_copy(
                rbuf[buf], out_hbm.at[pl.ds(base + c * _CH, _CH)], ss[buf])

        def s_wait(buf):
            pltpu.make_async_copy(
                rbuf[buf], out_hbm.at[pl.ds(base, _CH)], ss[buf]).wait()

        for b in range(nbuf):
            g_start(b, b)
        for c in range(nc):
            buf = c % nbuf
            g_wait(buf)
            s_start(c, buf)
            if c + nbuf < nc:
                s_wait(buf)
                g_start(c + nbuf, buf)
        for c in range(max(nc - nbuf, 0), nc):
            s_wait(c % nbuf)

    return gk(table, idx2)


_T1 = 4096
_T = 1024


def _p1_body(x_ref, logs_ref, bias_ref, w_ref, b1_ref, z_ref, a_ref, bm_ref):
    z = x_ref[...] * jnp.exp(logs_ref[...]) + bias_ref[...]
    z_ref[...] = z
    cat = jnp.dot(z[:, :C1], w_ref[...], preferred_element_type=jnp.float32)
    a_ref[...] = cat[:, :D2] + b1_ref[...]
    bm_ref[...] = cat[:, D2:]


def _coupling_core(g_ref, a_ref, z_ref, w2_ref, b2_ref, w3_ref, b3_ref):
    g = g_ref[...]
    a = a_ref[...]
    h1 = jax.nn.relu(g.reshape(_T, KNN, D2) + a[:, None, :]).reshape(_T * KNN, D2)
    h2 = jax.nn.relu(
        jnp.dot(h1, w2_ref[...], preferred_element_type=jnp.float32) + b2_ref[...])
    m = jnp.max(h2.reshape(_T, KNN, D2), axis=1)
    o = jnp.dot(m, w3_ref[...], preferred_element_type=jnp.float32) + b3_ref[...]
    sl = o[:, :C2] + 2.0
    s = jax.nn.sigmoid(sl)
    z = z_ref[...]
    xb2 = (z[:, C1:] + o[:, C2:]) * s
    z1 = jnp.concatenate([z[:, :C1], xb2], axis=1)
    part = -jnp.sum(jnp.maximum(-sl, 0.0) + jnp.log1p(jnp.exp(-jnp.abs(sl))))
    return z1, part


def _p3_body(g_ref, a_ref, z_ref, logs_ref, w2_ref, b2_ref, w3_ref, b3_ref,
             p_ref, w1n_ref, b1n_ref, zp_ref, a2_ref, bm2_ref, ld_ref):
    t_idx = pl.program_id(0)
    z1, part = _coupling_core(g_ref, a_ref, z_ref, w2_ref, b2_ref, w3_ref, b3_ref)
    zp = jnp.dot(z1, p_ref[...], preferred_element_type=jnp.float32)
    zp_ref[...] = zp
    cat = jnp.dot(zp[:, :C1], w1n_ref[...], preferred_element_type=jnp.float32)
    a2_ref[...] = cat[:, :D2] + b1n_ref[...]
    bm2_ref[...] = cat[:, D2:]

    @pl.when(t_idx == 0)
    def _():
        ld_ref[0, 0] = jnp.sum(logs_ref[...]) * NPTS + part

    @pl.when(t_idx != 0)
    def _():
        ld_ref[0, 0] += part


def _p5_body(g_ref, a_ref, z_ref, ld3_ref, w2_ref, b2_ref, w3_ref, b3_ref,
             zf_ref, ld_ref):
    t_idx = pl.program_id(0)
    z1, part = _coupling_core(g_ref, a_ref, z_ref, w2_ref, b2_ref, w3_ref, b3_ref)
    zf_ref[...] = z1

    @pl.when(t_idx == 0)
    def _():
        ld_ref[0, 0] = ld3_ref[0, 0] + part

    @pl.when(t_idx != 0)
    def _():
        ld_ref[0, 0] += part


def _fold_l1(p):
    w = p["l1"]["w"]
    wc, wd, wn = w[:C1], w[C1:2 * C1], w[2 * C1:]
    return wc - wd, wd + wn, p["l1"]["b"]


def _pack_coupling(pF, pG):
    waF, wbF, b1F = _fold_l1(pF)
    waG, wbG, b1G = _fold_l1(pG)
    w1 = jnp.concatenate([waF, waG, wbF, wbG], axis=1)
    b1 = jnp.concatenate([b1F, b1G])[None, :]
    w2 = jnp.zeros((D2, D2), jnp.float32)
    w2 = w2.at[:HDIM, :HDIM].set(pF["l2"]["w"]).at[HDIM:, HDIM:].set(pG["l2"]["w"])
    b2 = jnp.concatenate([pF["l2"]["b"], pG["l2"]["b"]])[None, :]
    w3 = jnp.zeros((D2, 2 * C2), jnp.float32)
    w3 = w3.at[:HDIM, :C2].set(pF["l3"]["w"]).at[HDIM:, C2:].set(pG["l3"]["w"])
    b3 = jnp.concatenate([pF["l3"]["b"], pG["l3"]["b"]])[None, :]
    return w1, b1, w2, b2, w3, b3


_SEM_ARB = pltpu.CompilerParams(dimension_semantics=("arbitrary",))


def _full2d(r, c):
    return pl.BlockSpec((r, c), lambda t: (0, 0))


def kernel(x, knn_idx, params, perm):
    x = x.astype(jnp.float32)
    idx2 = knn_idx.astype(jnp.int32).reshape(NB, (NPTS * KNN) // _CH, _CH)
    logs = params["actnorm"]["logs"].reshape(1, IDIM)
    bias = params["actnorm"]["bias"].reshape(1, IDIM)
    w1c1, b1c1, w2c1, b2c1, w3c1, b3c1 = _pack_coupling(params["c1F"], params["c1G"])
    w1c2, b1c2, w2c2, b2c2, w3c2, b3c2 = _pack_coupling(params["c2F"], params["c2G"])
    pmat = (jnp.arange(IDIM)[:, None] == perm[None, :].astype(jnp.int32)
            ).astype(jnp.float32)

    f32 = jnp.float32
    nt1 = NPTS // _T1
    nt = NPTS // _T

    p1 = pl.pallas_call(
        _p1_body,
        grid=(nt1,),
        in_specs=[
            pl.BlockSpec((_T1, IDIM), lambda t: (t, 0)),
            _full2d(1, IDIM), _full2d(1, IDIM),
            _full2d(C1, 2 * D2), _full2d(1, D2),
        ],
        out_specs=[
            pl.BlockSpec((_T1, IDIM), lambda t: (t, 0)),
            pl.BlockSpec((_T1, D2), lambda t: (t, 0)),
            pl.BlockSpec((_T1, D2), lambda t: (t, 0)),
        ],
        out_shape=[
            jax.ShapeDtypeStruct((NPTS, IDIM), f32),
            jax.ShapeDtypeStruct((NPTS, D2), f32),
            jax.ShapeDtypeStruct((NPTS, D2), f32),
        ],
        compiler_params=_SEM_ARB,
    )

    p3 = pl.pallas_call(
        _p3_body,
        grid=(nt,),
        in_specs=[
            pl.BlockSpec((_T * KNN, D2), lambda t: (t, 0)),
            pl.BlockSpec((_T, D2), lambda t: (t, 0)),
            pl.BlockSpec((_T, IDIM), lambda t: (t, 0)),
            _full2d(1, IDIM),
            _full2d(D2, D2), _full2d(1, D2),
            _full2d(D2, 2 * C2), _full2d(1, 2 * C2),
            _full2d(IDIM, IDIM),
            _full2d(C1, 2 * D2), _full2d(1, D2),
        ],
        out_specs=[
            pl.BlockSpec((_T, IDIM), lambda t: (t, 0)),
            pl.BlockSpec((_T, D2), lambda t: (t, 0)),
            pl.BlockSpec((_T, D2), lambda t: (t, 0)),
            pl.BlockSpec((1, 1), lambda t: (0, 0), memory_space=pltpu.SMEM),
        ],
        out_shape=[
            jax.ShapeDtypeStruct((NPTS, IDIM), f32),
            jax.ShapeDtypeStruct((NPTS, D2), f32),
            jax.ShapeDtypeStruct((NPTS, D2), f32),
            jax.ShapeDtypeStruct((1, 1), f32),
        ],
        compiler_params=_SEM_ARB,
    )

    p5 = pl.pallas_call(
        _p5_body,
        grid=(nt,),
        in_specs=[
            pl.BlockSpec((_T * KNN, D2), lambda t: (t, 0)),
            pl.BlockSpec((_T, D2), lambda t: (t, 0)),
            pl.BlockSpec((_T, IDIM), lambda t: (t, 0)),
            pl.BlockSpec((1, 1), lambda t: (0, 0), memory_space=pltpu.SMEM),
            _full2d(D2, D2), _full2d(1, D2),
            _full2d(D2, 2 * C2), _full2d(1, 2 * C2),
        ],
        out_specs=[
            pl.BlockSpec((_T, IDIM), lambda t: (t, 0)),
            pl.BlockSpec((1, 1), lambda t: (0, 0), memory_space=pltpu.SMEM),
        ],
        out_shape=[
            jax.ShapeDtypeStruct((NPTS, IDIM), f32),
            jax.ShapeDtypeStruct((1, 1), f32),
        ],
        compiler_params=_SEM_ARB,
    )

    st1 = [p1(x[b], logs, bias, w1c1, b1c1) for b in range(NB)]
    g1 = [_sc_gather(st1[b][2], idx2[b]) for b in range(NB)]
    st3 = [p3(g1[b], st1[b][1], st1[b][0], logs, w2c1, b2c1, w3c1, b3c1,
              pmat, w1c2, b1c2) for b in range(NB)]
    g2 = [_sc_gather(st3[b][2], idx2[b]) for b in range(NB)]
    st5 = [p5(g2[b], st3[b][1], st3[b][0], st3[b][3], w2c2, b2c2, w3c2, b3c2)
           for b in range(NB)]

    return (jnp.stack([st5[b][0] for b in range(NB)]),
            jnp.stack([st5[b][1][0, 0] for b in range(NB)]))
